# baseline (speedup 1.0000x reference)
import functools

import jax
import jax.numpy as jnp
from jax import lax
from jax.experimental import pallas as pl
from jax.experimental.pallas import tpu as pltpu
from jax.experimental.pallas import tpu_sc as plsc

N0 = 5000
N1 = 5000
N = N0 + N1
E = 320000
ER = E // 2
Q = 100000
IN = 128
HID = 128
OUT = 64

NC = 2
NS = 16
NW = NC * NS
K = 80
TRIPS = 128
CE = TRIPS * K
ERP = CE * NS
EP = 2 * ERP
U = 2
GROUPS = TRIPS // U
DUMMY = 640
CHUNK = 640
TAIL = N - (NS - 1) * CHUNK
CNT_CHUNK = 640
QP = 102400
QT = QP // NW
KD = 128
DTRIPS = QT // KD

_f32 = jnp.float32
_i32 = jnp.int32


def _m1_body(x0_ref, x1_ref, lw0_ref, lb0_ref, lw1_ref, lb1_ref, wrel_ref,
             wroot_ref, b1_ref, t_ref, root_ref):
    h0 = jnp.dot(x0_ref[...], lw0_ref[...], preferred_element_type=_f32) + lb0_ref[...]
    h1 = jnp.dot(x1_ref[...], lw1_ref[...], preferred_element_type=_f32) + lb1_ref[...]
    h = jnp.concatenate([h0, h1], axis=0)
    t_ref[...] = jnp.concatenate(
        [jnp.dot(h, wrel_ref[0], preferred_element_type=_f32),
         jnp.dot(h, wrel_ref[1], preferred_element_type=_f32)], axis=0)
    root_ref[...] = jnp.dot(h, wroot_ref[...], preferred_element_type=_f32) + b1_ref[...]


_m1 = pl.pallas_call(
    _m1_body,
    out_shape=[
        jax.ShapeDtypeStruct((2 * N, HID), _f32),
        jax.ShapeDtypeStruct((N, HID), _f32),
    ],
)


def _m2_body(root_ref, a0_ref, a1_ref, c0_ref, c1_ref, wrel_ref, wroot_ref,
             b_ref, tp_ref, root2_ref):
    inv0 = 1.0 / jnp.maximum(c0_ref[...], 1.0)
    inv1 = 1.0 / jnp.maximum(c1_ref[...], 1.0)
    z = root_ref[...] + a0_ref[...] * inv0[:, None] + a1_ref[...] * inv1[:, None]
    z = jnp.maximum(z, 0.0)
    tp_ref[...] = jnp.concatenate(
        [jnp.dot(z, wrel_ref[0], preferred_element_type=_f32),
         jnp.dot(z, wrel_ref[1], preferred_element_type=_f32)], axis=1)
    root2_ref[...] = jnp.dot(z, wroot_ref[...], preferred_element_type=_f32) + b_ref[...]


_m2 = pl.pallas_call(
    _m2_body,
    out_shape=[
        jax.ShapeDtypeStruct((N, 2 * OUT), _f32),
        jax.ShapeDtypeStruct((N, OUT), _f32),
    ],
)


def _m3_body(root_ref, a0_ref, a1_ref, c0_ref, c1_ref, wuv_ref, buv_ref, uv_ref):
    inv0 = 1.0 / jnp.maximum(c0_ref[...], 1.0)
    inv1 = 1.0 / jnp.maximum(c1_ref[...], 1.0)
    a0 = a0_ref[...][:, :OUT]
    a1 = a1_ref[...][:, OUT:]
    z = root_ref[...] + a0 * inv0[:, None] + a1 * inv1[:, None]
    uv_ref[...] = jnp.dot(z, wuv_ref[...], preferred_element_type=_f32) + buv_ref[...]


_m3 = pl.pallas_call(
    _m3_body,
    out_shape=jax.ShapeDtypeStruct((N, 2), _f32),
)


def _fill_vec(ref, n, value):
    def body(j, carry):
        ref[pl.ds(j * 16, 16)] = jnp.full((16,), value, _f32)
        return carry
    lax.fori_loop(0, n // 16, body, 0)


def _make_agg(d, with_counts):
    mesh = plsc.VectorSubcoreMesh(
        core_axis_name="c", subcore_axis_name="s", num_cores=NC, num_subcores=NS)
    out_type = [
        jax.ShapeDtypeStruct((N, d), _f32),
        jax.ShapeDtypeStruct((N, d), _f32),
    ]
    scratch = [
        pltpu.VMEM((K,), _i32), pltpu.VMEM((K,), _i32),
        pltpu.VMEM((K,), _i32), pltpu.VMEM((K,), _i32),
        pltpu.VMEM((K, d), _f32), pltpu.VMEM((K, d), _f32),
        pltpu.VMEM_SHARED((N + DUMMY, d), _f32),
        pltpu.SemaphoreType.DMA((U,)),
        pltpu.SemaphoreType.DMA((U,)),
        pltpu.SemaphoreType.DMA((U,)),
        pltpu.SemaphoreType.DMA((U,)),
    ]
    if with_counts:
        out_type += [
            jax.ShapeDtypeStruct((N,), _f32),
            jax.ShapeDtypeStruct((N,), _f32),
        ]
        scratch += [
            pltpu.VMEM((K,), _f32),
            pltpu.VMEM((CNT_CHUNK,), _f32),
            pltpu.VMEM_SHARED((N + DUMMY,), _f32),
            pltpu.SemaphoreType.DMA((U,)),
        ]

    def body(table_hbm, srcp_hbm, dst_hbm, zeros_hbm, agg0_out, agg1_out, *rest):
        if with_counts:
            (cnt0_out, cnt1_out, sa_v, sb_v, da_v, db_v, ra_v, rb_v, acc_sh,
             isem, dsem, gsem, ssem, ones_v, zcnt_v, cnt_sh, csem) = rest
        else:
            (sa_v, sb_v, da_v, db_v, ra_v, rb_v, acc_sh,
             isem, dsem, gsem, ssem) = rest
        sidx = [sa_v, sb_v]
        didx = [da_v, db_v]
        rows = [ra_v, rb_v]
        c = lax.axis_index("c")
        s = lax.axis_index("s")
        base_e = c * ERP + s * CE

        @pl.when(s < NS - 1)
        def _():
            pltpu.sync_copy(zeros_hbm, acc_sh.at[pl.ds(s * CHUNK, CHUNK)])

        @pl.when(s == NS - 1)
        def _():
            pltpu.sync_copy(zeros_hbm.at[pl.ds(0, TAIL)],
                            acc_sh.at[pl.ds((NS - 1) * CHUNK, TAIL)])

        if with_counts:
            _fill_vec(ones_v, K, 1.0)
            _fill_vec(zcnt_v, CNT_CHUNK, 0.0)

            @pl.when(s < NS - 1)
            def _():
                pltpu.sync_copy(zcnt_v, cnt_sh.at[pl.ds(s * CNT_CHUNK, CNT_CHUNK)])

            @pl.when(s == NS - 1)
            def _():
                pltpu.sync_copy(zcnt_v.at[pl.ds(0, N - (NS - 1) * CNT_CHUNK)],
                                cnt_sh.at[pl.ds((NS - 1) * CNT_CHUNK,
                                                N - (NS - 1) * CNT_CHUNK)])
        plsc.subcore_barrier()

        def trip(t, carry):
            off = pl.multiple_of(base_e + t * K, 8)
            pltpu.sync_copy(srcp_hbm.at[pl.ds(off, K)], sidx[0])
            pltpu.sync_copy(dst_hbm.at[pl.ds(off, K)], didx[0])
            pltpu.sync_copy(table_hbm.at[sidx[0]], rows[0])
            pltpu.sync_copy(rows[0], acc_sh.at[didx[0]], add=True)
            if with_counts:
                pltpu.sync_copy(ones_v, cnt_sh.at[didx[0]], add=True)
            return carry

        lax.fori_loop(0, TRIPS, trip, 0)
        plsc.subcore_barrier()

        for cc, agg_out in ((0, agg0_out), (1, agg1_out)):
            @pl.when(jnp.logical_and(c == cc, s < NS - 1))
            def _(agg_out=agg_out):
                pltpu.sync_copy(acc_sh.at[pl.ds(s * CHUNK, CHUNK)],
                                agg_out.at[pl.ds(s * CHUNK, CHUNK)])

            @pl.when(jnp.logical_and(c == cc, s == NS - 1))
            def _(agg_out=agg_out):
                pltpu.sync_copy(acc_sh.at[pl.ds((NS - 1) * CHUNK, TAIL)],
                                agg_out.at[pl.ds((NS - 1) * CHUNK, TAIL)])

        if with_counts:
            tail = N - (NS - 1) * CNT_CHUNK
            for cc, cnt_out in ((0, cnt0_out), (1, cnt1_out)):
                @pl.when(jnp.logical_and(c == cc, s < NS - 1))
                def _(cnt_out=cnt_out):
                    pltpu.sync_copy(cnt_sh.at[pl.ds(s * CNT_CHUNK, CNT_CHUNK)], zcnt_v)
                    pltpu.sync_copy(zcnt_v, cnt_out.at[pl.ds(s * CNT_CHUNK, CNT_CHUNK)])

                @pl.when(jnp.logical_and(c == cc, s == NS - 1))
                def _(cnt_out=cnt_out):
                    pltpu.sync_copy(cnt_sh.at[pl.ds((NS - 1) * CNT_CHUNK, tail)],
                                    zcnt_v.at[pl.ds(0, tail)])
                    pltpu.sync_copy(zcnt_v.at[pl.ds(0, tail)],
                                    cnt_out.at[pl.ds((NS - 1) * CNT_CHUNK, tail)])

    return pl.kernel(body, out_type=out_type, mesh=mesh, scratch_types=scratch)


_agg_cache = functools.lru_cache(maxsize=None)(_make_agg)


def _make_decode():
    mesh = plsc.VectorSubcoreMesh(
        core_axis_name="c", subcore_axis_name="s", num_cores=NC, num_subcores=NS)
    out_type = jax.ShapeDtypeStruct((QP,), _f32)
    scratch = [
        pltpu.VMEM((N,), _f32),
        pltpu.VMEM((N,), _f32),
        pltpu.VMEM((QT,), _i32),
        pltpu.VMEM((QT,), _i32),
        pltpu.VMEM((KD,), _f32),
    ]

    def body(u_hbm, v_hbm, i0_hbm, i1_hbm, out_hbm, u_v, v_v, i0_v, i1_v, r_v):
        c = lax.axis_index("c")
        s = lax.axis_index("s")
        w = s * NC + c
        base = w * QT
        pltpu.sync_copy(u_hbm, u_v)
        pltpu.sync_copy(v_hbm, v_v)
        pltpu.sync_copy(i0_hbm.at[pl.ds(base, QT)], i0_v)
        pltpu.sync_copy(i1_hbm.at[pl.ds(base, QT)], i1_v)

        def step(i, carry):
            for j in range(KD // 16):
                a = plsc.load_gather(u_v, [i0_v[pl.ds(i * KD + j * 16, 16)]])
                b = plsc.load_gather(v_v, [i1_v[pl.ds(i * KD + j * 16, 16)]])
                x = a + b
                r_v[pl.ds(j * 16, 16)] = 1.0 / (1.0 + jnp.exp(-x))
            off = pl.multiple_of(base + i * KD, 8)
            pltpu.sync_copy(r_v, out_hbm.at[pl.ds(off, KD)])
            return carry

        lax.fori_loop(0, DTRIPS, step, 0)

    return pl.kernel(body, out_type=out_type, mesh=mesh, scratch_types=scratch,
                     compiler_params=pltpu.CompilerParams(
                         use_tc_tiling_on_sc=False, needs_layout_passes=False))


_decode_cache = functools.lru_cache(maxsize=None)(_make_decode)


def kernel(x0, x1, edge_index, index, lin0_W, lin0_b, lin1_W, lin1_b,
           W1_rel, W1_root, b1, W2_rel, W2_root, b2, fc_W, fc_b):
    src = jnp.asarray(edge_index[0], _i32)
    dst = jnp.asarray(edge_index[1], _i32)
    i0 = jnp.asarray(index[0], _i32)
    i1 = jnp.asarray(index[1], _i32)

    npad = ERP - ER
    pad0 = jnp.zeros((npad,), _i32)
    padN = N + (jnp.arange(npad, dtype=_i32) % DUMMY)
    srcp1 = jnp.concatenate([src[:ER], pad0, src[ER:] + N, pad0])
    srcp2 = jnp.concatenate([src[:ER], pad0, src[ER:], pad0])
    dstp = jnp.concatenate([dst[:ER], padN, dst[ER:], padN])
    i0p = jnp.concatenate([i0, jnp.zeros((QP - Q,), _i32)])
    i1p = jnp.concatenate([i1, jnp.zeros((QP - Q,), _i32)])

    t1, root1 = _m1(
        x0, x1, lin0_W, lin0_b.reshape(1, IN), lin1_W, lin1_b.reshape(1, IN),
        W1_rel, W1_root, b1.reshape(1, HID))
    zrows = jnp.zeros((CHUNK, HID), _f32)
    agg1_0, agg1_1, cnt0, cnt1 = _agg_cache(HID, True)(t1, srcp1, dstp, zrows)
    t2p, root2 = _m2(
        root1, agg1_0, agg1_1, cnt0, cnt1, W2_rel, W2_root, b2.reshape(1, OUT))
    agg2_0, agg2_1 = _agg_cache(2 * OUT, False)(t2p, srcp2, dstp, zrows)

    wuv = jnp.concatenate([fc_W[:OUT], fc_W[OUT:]], axis=1)
    buv = jnp.concatenate([fc_b, jnp.zeros((1,), _f32)]).reshape(1, 2)
    uv = _m3(root2, agg2_0, agg2_1, cnt0, cnt1, wuv, buv)
    out = _decode_cache()(uv[:, 0], uv[:, 1], i0p, i1p)
    return out[:Q].reshape(Q, 1)


# speedup vs baseline: 1.6601x; 1.6601x over previous
import functools

import jax
import jax.numpy as jnp
from jax import lax
from jax.experimental import pallas as pl
from jax.experimental.pallas import tpu as pltpu
from jax.experimental.pallas import tpu_sc as plsc

N0 = 5000
N1 = 5000
N = N0 + N1
E = 320000
ER = E // 2
Q = 100000
IN = 128
HID = 128
OUT = 64

NC = 2
NS = 16
NW = NC * NS
K = 80
CE = ER // NS
TRIPS = CE // K
ZROWS = 80
CHUNK = 640
TAIL = N - (NS - 1) * CHUNK
CNT_CHUNK = 640
QP = 102400
QT = QP // NW
KD = 128
DTRIPS = QT // KD

_f32 = jnp.float32
_i32 = jnp.int32


def _m1_body(x0_ref, x1_ref, lw0_ref, lb0_ref, lw1_ref, lb1_ref, wrel_ref,
             wroot_ref, b1_ref, t0_ref, t1_ref, root_ref):
    h0 = jnp.dot(x0_ref[...], lw0_ref[...], preferred_element_type=_f32) + lb0_ref[...]
    h1 = jnp.dot(x1_ref[...], lw1_ref[...], preferred_element_type=_f32) + lb1_ref[...]
    h = jnp.concatenate([h0, h1], axis=0)
    t0_ref[...] = jnp.dot(h, wrel_ref[0], preferred_element_type=_f32)
    t1_ref[...] = jnp.dot(h, wrel_ref[1], preferred_element_type=_f32)
    root_ref[...] = jnp.dot(h, wroot_ref[...], preferred_element_type=_f32) + b1_ref[...]


_m1 = pl.pallas_call(
    _m1_body,
    out_shape=[
        jax.ShapeDtypeStruct((N, HID), _f32),
        jax.ShapeDtypeStruct((N, HID), _f32),
        jax.ShapeDtypeStruct((N, HID), _f32),
    ],
)


def _m2_body(root_ref, a0_ref, a1_ref, c0_ref, c1_ref, wrel_ref, wroot_ref,
             b_ref, tp_ref, root2_ref):
    inv0 = 1.0 / jnp.maximum(c0_ref[...], 1.0)
    inv1 = 1.0 / jnp.maximum(c1_ref[...], 1.0)
    z = root_ref[...] + a0_ref[...] * inv0[:, None] + a1_ref[...] * inv1[:, None]
    z = jnp.maximum(z, 0.0)
    tp_ref[...] = jnp.concatenate(
        [jnp.dot(z, wrel_ref[0], preferred_element_type=_f32),
         jnp.dot(z, wrel_ref[1], preferred_element_type=_f32)], axis=1)
    root2_ref[...] = jnp.dot(z, wroot_ref[...], preferred_element_type=_f32) + b_ref[...]


_m2 = pl.pallas_call(
    _m2_body,
    out_shape=[
        jax.ShapeDtypeStruct((N, 2 * OUT), _f32),
        jax.ShapeDtypeStruct((N, OUT), _f32),
    ],
)


def _m3_body(root_ref, a0_ref, a1_ref, c0_ref, c1_ref, wuv_ref, buv_ref, uv_ref):
    inv0 = 1.0 / jnp.maximum(c0_ref[...], 1.0)
    inv1 = 1.0 / jnp.maximum(c1_ref[...], 1.0)
    a0 = a0_ref[...][:, :OUT]
    a1 = a1_ref[...][:, OUT:]
    z = root_ref[...] + a0 * inv0[:, None] + a1 * inv1[:, None]
    uv_ref[...] = jnp.dot(z, wuv_ref[...], preferred_element_type=_f32) + buv_ref[...]


_m3 = pl.pallas_call(
    _m3_body,
    out_shape=jax.ShapeDtypeStruct((N, 2), _f32),
)


def _zero_rows(ref, rows, d):
    def row_body(r, carry):
        def col_body(j, carry2):
            ref[r, pl.ds(j * 16, 16)] = jnp.zeros((16,), _f32)
            return carry2
        return lax.fori_loop(0, d // 16, col_body, carry)
    lax.fori_loop(0, rows, row_body, 0)


def _fill_vec(ref, n, value):
    def body(j, carry):
        ref[pl.ds(j * 16, 16)] = jnp.full((16,), value, _f32)
        return carry
    lax.fori_loop(0, n // 16, body, 0)


def _make_agg(d, with_counts):
    mesh = plsc.VectorSubcoreMesh(
        core_axis_name="c", subcore_axis_name="s", num_cores=NC, num_subcores=NS)
    out_type = [
        jax.ShapeDtypeStruct((N, d), _f32),
        jax.ShapeDtypeStruct((N, d), _f32),
    ]
    scratch = [
        pltpu.VMEM((K,), _i32),
        pltpu.VMEM((K,), _i32),
        pltpu.VMEM((K, d), _f32),
        pltpu.VMEM((ZROWS, d), _f32),
        pltpu.VMEM_SHARED((N, d), _f32),
    ]
    if with_counts:
        out_type += [
            jax.ShapeDtypeStruct((N,), _f32),
            jax.ShapeDtypeStruct((N,), _f32),
        ]
        scratch += [
            pltpu.VMEM((K,), _f32),
            pltpu.VMEM((CNT_CHUNK,), _f32),
            pltpu.VMEM_SHARED((N,), _f32),
        ]

    def body(t0_hbm, t1_hbm, src_hbm, dst_hbm, agg0_out, agg1_out, *rest):
        if with_counts:
            (cnt0_out, cnt1_out, sidx_v, didx_v, rows_v, zrows_v, acc_sh,
             ones_v, zcnt_v, cnt_sh) = rest
        else:
            sidx_v, didx_v, rows_v, zrows_v, acc_sh = rest
        c = lax.axis_index("c")
        s = lax.axis_index("s")
        base_e = c * ER + s * CE

        _zero_rows(zrows_v, ZROWS, d)

        @pl.when(s < NS - 1)
        def _():
            for kk in range(CHUNK // ZROWS):
                pltpu.sync_copy(zrows_v, acc_sh.at[pl.ds(s * CHUNK + kk * ZROWS, ZROWS)])

        @pl.when(s == NS - 1)
        def _():
            for kk in range(TAIL // ZROWS):
                pltpu.sync_copy(zrows_v, acc_sh.at[pl.ds((NS - 1) * CHUNK + kk * ZROWS, ZROWS)])

        if with_counts:
            _fill_vec(ones_v, K, 1.0)
            _fill_vec(zcnt_v, CNT_CHUNK, 0.0)

            @pl.when(s < NS - 1)
            def _():
                pltpu.sync_copy(zcnt_v, cnt_sh.at[pl.ds(s * CNT_CHUNK, CNT_CHUNK)])

            @pl.when(s == NS - 1)
            def _():
                pltpu.sync_copy(zcnt_v.at[pl.ds(0, N - (NS - 1) * CNT_CHUNK)],
                                cnt_sh.at[pl.ds((NS - 1) * CNT_CHUNK,
                                                N - (NS - 1) * CNT_CHUNK)])
        plsc.subcore_barrier()

        def trip(t, carry):
            off = pl.multiple_of(base_e + t * K, 8)
            pltpu.sync_copy(src_hbm.at[pl.ds(off, K)], sidx_v)
            pltpu.sync_copy(dst_hbm.at[pl.ds(off, K)], didx_v)

            @pl.when(c == 0)
            def _():
                pltpu.sync_copy(t0_hbm.at[sidx_v], rows_v)

            @pl.when(c == 1)
            def _():
                pltpu.sync_copy(t1_hbm.at[sidx_v], rows_v)

            pltpu.sync_copy(rows_v, acc_sh.at[didx_v], add=True)
            if with_counts:
                pltpu.sync_copy(ones_v, cnt_sh.at[didx_v], add=True)
            return carry

        lax.fori_loop(0, TRIPS, trip, 0)
        plsc.subcore_barrier()

        for cc, agg_out in ((0, agg0_out), (1, agg1_out)):
            @pl.when(jnp.logical_and(c == cc, s < NS - 1))
            def _(agg_out=agg_out):
                pltpu.sync_copy(acc_sh.at[pl.ds(s * CHUNK, CHUNK)],
                                agg_out.at[pl.ds(s * CHUNK, CHUNK)])

            @pl.when(jnp.logical_and(c == cc, s == NS - 1))
            def _(agg_out=agg_out):
                pltpu.sync_copy(acc_sh.at[pl.ds((NS - 1) * CHUNK, TAIL)],
                                agg_out.at[pl.ds((NS - 1) * CHUNK, TAIL)])

        if with_counts:
            tail = N - (NS - 1) * CNT_CHUNK
            for cc, cnt_out in ((0, cnt0_out), (1, cnt1_out)):
                @pl.when(jnp.logical_and(c == cc, s < NS - 1))
                def _(cnt_out=cnt_out):
                    pltpu.sync_copy(cnt_sh.at[pl.ds(s * CNT_CHUNK, CNT_CHUNK)], zcnt_v)
                    pltpu.sync_copy(zcnt_v, cnt_out.at[pl.ds(s * CNT_CHUNK, CNT_CHUNK)])

                @pl.when(jnp.logical_and(c == cc, s == NS - 1))
                def _(cnt_out=cnt_out):
                    pltpu.sync_copy(cnt_sh.at[pl.ds((NS - 1) * CNT_CHUNK, tail)],
                                    zcnt_v.at[pl.ds(0, tail)])
                    pltpu.sync_copy(zcnt_v.at[pl.ds(0, tail)],
                                    cnt_out.at[pl.ds((NS - 1) * CNT_CHUNK, tail)])

    return pl.kernel(body, out_type=out_type, mesh=mesh, scratch_types=scratch)


_agg_cache = functools.lru_cache(maxsize=None)(_make_agg)


def _make_decode():
    mesh = plsc.VectorSubcoreMesh(
        core_axis_name="c", subcore_axis_name="s", num_cores=NC, num_subcores=NS)
    out_type = jax.ShapeDtypeStruct((QP,), _f32)
    scratch = [
        pltpu.VMEM((N,), _f32),
        pltpu.VMEM((N,), _f32),
        pltpu.VMEM((QT,), _i32),
        pltpu.VMEM((QT,), _i32),
        pltpu.VMEM((KD,), _f32),
    ]

    def body(u_hbm, v_hbm, i0_hbm, i1_hbm, out_hbm, u_v, v_v, i0_v, i1_v, r_v):
        c = lax.axis_index("c")
        s = lax.axis_index("s")
        w = s * NC + c
        base = w * QT
        pltpu.sync_copy(u_hbm, u_v)
        pltpu.sync_copy(v_hbm, v_v)
        pltpu.sync_copy(i0_hbm.at[pl.ds(base, QT)], i0_v)
        pltpu.sync_copy(i1_hbm.at[pl.ds(base, QT)], i1_v)

        def step(i, carry):
            for j in range(KD // 16):
                a = plsc.load_gather(u_v, [i0_v[pl.ds(i * KD + j * 16, 16)]])
                b = plsc.load_gather(v_v, [i1_v[pl.ds(i * KD + j * 16, 16)]])
                x = a + b
                r_v[pl.ds(j * 16, 16)] = 1.0 / (1.0 + jnp.exp(-x))
            off = pl.multiple_of(base + i * KD, 8)
            pltpu.sync_copy(r_v, out_hbm.at[pl.ds(off, KD)])
            return carry

        lax.fori_loop(0, DTRIPS, step, 0)

    return pl.kernel(body, out_type=out_type, mesh=mesh, scratch_types=scratch,
                     compiler_params=pltpu.CompilerParams(
                         use_tc_tiling_on_sc=False, needs_layout_passes=False))


_decode_cache = functools.lru_cache(maxsize=None)(_make_decode)


def kernel(x0, x1, edge_index, index, lin0_W, lin0_b, lin1_W, lin1_b,
           W1_rel, W1_root, b1, W2_rel, W2_root, b2, fc_W, fc_b):
    src = jnp.asarray(edge_index[0], _i32)
    dst = jnp.asarray(edge_index[1], _i32)
    i0 = jnp.asarray(index[0], _i32)
    i1 = jnp.asarray(index[1], _i32)

    i0p = jnp.concatenate([i0, jnp.zeros((QP - Q,), _i32)])
    i1p = jnp.concatenate([i1, jnp.zeros((QP - Q,), _i32)])

    t1_0, t1_1, root1 = _m1(
        x0, x1, lin0_W, lin0_b.reshape(1, IN), lin1_W, lin1_b.reshape(1, IN),
        W1_rel, W1_root, b1.reshape(1, HID))
    agg1_0, agg1_1, cnt0, cnt1 = _agg_cache(HID, True)(t1_0, t1_1, src, dst)
    t2p, root2 = _m2(
        root1, agg1_0, agg1_1, cnt0, cnt1, W2_rel, W2_root, b2.reshape(1, OUT))
    agg2_0, agg2_1 = _agg_cache(2 * OUT, False)(t2p, t2p, src, dst)

    wuv = jnp.concatenate([fc_W[:OUT], fc_W[OUT:]], axis=1)
    buv = jnp.concatenate([fc_b, jnp.zeros((1,), _f32)]).reshape(1, 2)
    uv = _m3(root2, agg2_0, agg2_1, cnt0, cnt1, wuv, buv)
    out = _decode_cache()(uv[:, 0], uv[:, 1], i0p, i1p)
    return out[:Q].reshape(Q, 1)


# speedup vs baseline: 1.9653x; 1.1839x over previous
import functools

import jax
import jax.numpy as jnp
from jax import lax
from jax.experimental import pallas as pl
from jax.experimental.pallas import tpu as pltpu
from jax.experimental.pallas import tpu_sc as plsc

N0 = 5000
N1 = 5000
N = N0 + N1
E = 320000
ER = E // 2
Q = 100000
IN = 128
HID = 128
OUT = 64

NC = 2
NS = 16
NW = NC * NS
K = 80
CE = ER // NS
TRIPS = CE // K
ZROWS = 80
CHUNK = 640
TAIL = N - (NS - 1) * CHUNK
CNT_CHUNK = 640
QP = 102400
QT = QP // NW
KD = 128
DTRIPS = QT // KD

_f32 = jnp.float32
_i32 = jnp.int32


def _m1_body(x0_ref, x1_ref, lw0_ref, lb0_ref, lw1_ref, lb1_ref, wrel_ref,
             wroot_ref, b1_ref, t0_ref, t1_ref, root_ref):
    h0 = jnp.dot(x0_ref[...], lw0_ref[...], preferred_element_type=_f32) + lb0_ref[...]
    h1 = jnp.dot(x1_ref[...], lw1_ref[...], preferred_element_type=_f32) + lb1_ref[...]
    h = jnp.concatenate([h0, h1], axis=0)
    t0_ref[...] = jnp.dot(h, wrel_ref[0], preferred_element_type=_f32)
    t1_ref[...] = jnp.dot(h, wrel_ref[1], preferred_element_type=_f32)
    root_ref[...] = jnp.dot(h, wroot_ref[...], preferred_element_type=_f32) + b1_ref[...]


_m1 = pl.pallas_call(
    _m1_body,
    out_shape=[
        jax.ShapeDtypeStruct((N, HID), _f32),
        jax.ShapeDtypeStruct((N, HID), _f32),
        jax.ShapeDtypeStruct((N, HID), _f32),
    ],
)


def _m2_body(root_ref, a0_ref, a1_ref, c0_ref, c1_ref, wrel_ref, wroot_ref,
             b_ref, tp_ref, root2_ref):
    inv0 = 1.0 / jnp.maximum(c0_ref[...], 1.0)
    inv1 = 1.0 / jnp.maximum(c1_ref[...], 1.0)
    z = root_ref[...] + a0_ref[...] * inv0[:, None] + a1_ref[...] * inv1[:, None]
    z = jnp.maximum(z, 0.0)
    tp_ref[...] = jnp.concatenate(
        [jnp.dot(z, wrel_ref[0], preferred_element_type=_f32),
         jnp.dot(z, wrel_ref[1], preferred_element_type=_f32)], axis=1)
    root2_ref[...] = jnp.dot(z, wroot_ref[...], preferred_element_type=_f32) + b_ref[...]


_m2 = pl.pallas_call(
    _m2_body,
    out_shape=[
        jax.ShapeDtypeStruct((N, 2 * OUT), _f32),
        jax.ShapeDtypeStruct((N, OUT), _f32),
    ],
)


def _m3_body(root_ref, a0_ref, a1_ref, c0_ref, c1_ref, wuv_ref, buv_ref, uv_ref):
    inv0 = 1.0 / jnp.maximum(c0_ref[...], 1.0)
    inv1 = 1.0 / jnp.maximum(c1_ref[...], 1.0)
    a0 = a0_ref[...][:, :OUT]
    a1 = a1_ref[...][:, OUT:]
    z = root_ref[...] + a0 * inv0[:, None] + a1 * inv1[:, None]
    uv_ref[...] = jnp.dot(z, wuv_ref[...], preferred_element_type=_f32) + buv_ref[...]


_m3 = pl.pallas_call(
    _m3_body,
    out_shape=jax.ShapeDtypeStruct((N, 2), _f32),
)


def _zero_rows(ref, rows, d):
    def row_body(r, carry):
        def col_body(j, carry2):
            ref[r, pl.ds(j * 16, 16)] = jnp.zeros((16,), _f32)
            return carry2
        return lax.fori_loop(0, d // 16, col_body, carry)
    lax.fori_loop(0, rows, row_body, 0)


def _fill_vec(ref, n, value):
    def body(j, carry):
        ref[pl.ds(j * 16, 16)] = jnp.full((16,), value, _f32)
        return carry
    lax.fori_loop(0, n // 16, body, 0)


def _make_agg(d, with_counts):
    mesh = plsc.VectorSubcoreMesh(
        core_axis_name="c", subcore_axis_name="s", num_cores=NC, num_subcores=NS)
    out_type = [
        jax.ShapeDtypeStruct((N, d), _f32),
        jax.ShapeDtypeStruct((N, d), _f32),
    ]
    scratch = [
        pltpu.VMEM((K,), _i32), pltpu.VMEM((K,), _i32),
        pltpu.VMEM((K,), _i32), pltpu.VMEM((K,), _i32),
        pltpu.VMEM((K, d), _f32), pltpu.VMEM((K, d), _f32),
        pltpu.VMEM((ZROWS, d), _f32),
        pltpu.VMEM_SHARED((N, d), _f32),
        pltpu.SemaphoreType.DMA, pltpu.SemaphoreType.DMA,
    ]
    if with_counts:
        out_type += [
            jax.ShapeDtypeStruct((N,), _f32),
            jax.ShapeDtypeStruct((N,), _f32),
        ]
        scratch += [
            pltpu.VMEM((K,), _f32),
            pltpu.VMEM((CNT_CHUNK,), _f32),
            pltpu.VMEM_SHARED((N,), _f32),
            pltpu.SemaphoreType.DMA, pltpu.SemaphoreType.DMA,
        ]

    def body(t0_hbm, t1_hbm, src_hbm, dst_hbm, agg0_out, agg1_out, *rest):
        if with_counts:
            (cnt0_out, cnt1_out, sa_v, sb_v, da_v, db_v, ra_v, rb_v,
             zrows_v, acc_sh, ssemA, ssemB,
             ones_v, zcnt_v, cnt_sh, csemA, csemB) = rest
        else:
            (sa_v, sb_v, da_v, db_v, ra_v, rb_v,
             zrows_v, acc_sh, ssemA, ssemB) = rest
        c = lax.axis_index("c")
        s = lax.axis_index("s")
        base_e = c * ER + s * CE

        _zero_rows(zrows_v, ZROWS, d)

        @pl.when(s < NS - 1)
        def _():
            for kk in range(CHUNK // ZROWS):
                pltpu.sync_copy(zrows_v, acc_sh.at[pl.ds(s * CHUNK + kk * ZROWS, ZROWS)])

        @pl.when(s == NS - 1)
        def _():
            for kk in range(TAIL // ZROWS):
                pltpu.sync_copy(zrows_v, acc_sh.at[pl.ds((NS - 1) * CHUNK + kk * ZROWS, ZROWS)])

        if with_counts:
            _fill_vec(ones_v, K, 1.0)
            _fill_vec(zcnt_v, CNT_CHUNK, 0.0)

            @pl.when(s < NS - 1)
            def _():
                pltpu.sync_copy(zcnt_v, cnt_sh.at[pl.ds(s * CNT_CHUNK, CNT_CHUNK)])

            @pl.when(s == NS - 1)
            def _():
                pltpu.sync_copy(zcnt_v.at[pl.ds(0, N - (NS - 1) * CNT_CHUNK)],
                                cnt_sh.at[pl.ds((NS - 1) * CNT_CHUNK,
                                                N - (NS - 1) * CNT_CHUNK)])
        plsc.subcore_barrier()

        def load_and_gather(t, sidx_v, didx_v, rows_v):
            off = pl.multiple_of(base_e + t * K, 8)
            pltpu.sync_copy(src_hbm.at[pl.ds(off, K)], sidx_v)
            pltpu.sync_copy(dst_hbm.at[pl.ds(off, K)], didx_v)

            @pl.when(c == 0)
            def _():
                pltpu.sync_copy(t0_hbm.at[sidx_v], rows_v)

            @pl.when(c == 1)
            def _():
                pltpu.sync_copy(t1_hbm.at[sidx_v], rows_v)

        def issue_scatter(didx_v, rows_v, ssem, csem):
            descs = [pltpu.async_copy(rows_v, acc_sh.at[didx_v], ssem, add=True)]
            if with_counts:
                descs.append(
                    pltpu.async_copy(ones_v, cnt_sh.at[didx_v], csem, add=True))
            return descs

        def drain_b():
            pltpu.make_async_copy(t0_hbm.at[pl.ds(0, K)], rb_v, ssemB).wait()
            if with_counts:
                pltpu.make_async_copy(cnt0_out.at[pl.ds(0, K)], ones_v, csemB).wait()

        def group(g, carry):
            load_and_gather(2 * g, sa_v, da_v, ra_v)

            @pl.when(g > 0)
            def _():
                drain_b()

            da = issue_scatter(da_v, ra_v, ssemA, csemA if with_counts else None)
            load_and_gather(2 * g + 1, sb_v, db_v, rb_v)
            for dsc in da:
                dsc.wait()
            issue_scatter(db_v, rb_v, ssemB, csemB if with_counts else None)
            return carry

        lax.fori_loop(0, TRIPS // 2, group, 0)
        drain_b()
        if TRIPS % 2:
            load_and_gather(TRIPS - 1, sa_v, da_v, ra_v)
            for dsc in issue_scatter(da_v, ra_v, ssemA, csemA if with_counts else None):
                dsc.wait()
        plsc.subcore_barrier()

        for cc, agg_out in ((0, agg0_out), (1, agg1_out)):
            @pl.when(jnp.logical_and(c == cc, s < NS - 1))
            def _(agg_out=agg_out):
                pltpu.sync_copy(acc_sh.at[pl.ds(s * CHUNK, CHUNK)],
                                agg_out.at[pl.ds(s * CHUNK, CHUNK)])

            @pl.when(jnp.logical_and(c == cc, s == NS - 1))
            def _(agg_out=agg_out):
                pltpu.sync_copy(acc_sh.at[pl.ds((NS - 1) * CHUNK, TAIL)],
                                agg_out.at[pl.ds((NS - 1) * CHUNK, TAIL)])

        if with_counts:
            tail = N - (NS - 1) * CNT_CHUNK
            for cc, cnt_out in ((0, cnt0_out), (1, cnt1_out)):
                @pl.when(jnp.logical_and(c == cc, s < NS - 1))
                def _(cnt_out=cnt_out):
                    pltpu.sync_copy(cnt_sh.at[pl.ds(s * CNT_CHUNK, CNT_CHUNK)], zcnt_v)
                    pltpu.sync_copy(zcnt_v, cnt_out.at[pl.ds(s * CNT_CHUNK, CNT_CHUNK)])

                @pl.when(jnp.logical_and(c == cc, s == NS - 1))
                def _(cnt_out=cnt_out):
                    pltpu.sync_copy(cnt_sh.at[pl.ds((NS - 1) * CNT_CHUNK, tail)],
                                    zcnt_v.at[pl.ds(0, tail)])
                    pltpu.sync_copy(zcnt_v.at[pl.ds(0, tail)],
                                    cnt_out.at[pl.ds((NS - 1) * CNT_CHUNK, tail)])

    return pl.kernel(body, out_type=out_type, mesh=mesh, scratch_types=scratch)


_agg_cache = functools.lru_cache(maxsize=None)(_make_agg)


def _make_decode():
    mesh = plsc.VectorSubcoreMesh(
        core_axis_name="c", subcore_axis_name="s", num_cores=NC, num_subcores=NS)
    out_type = jax.ShapeDtypeStruct((QP,), _f32)
    scratch = [
        pltpu.VMEM((N,), _f32),
        pltpu.VMEM((N,), _f32),
        pltpu.VMEM((QT,), _i32),
        pltpu.VMEM((QT,), _i32),
        pltpu.VMEM((KD,), _f32),
    ]

    def body(u_hbm, v_hbm, i0_hbm, i1_hbm, out_hbm, u_v, v_v, i0_v, i1_v, r_v):
        c = lax.axis_index("c")
        s = lax.axis_index("s")
        w = s * NC + c
        base = w * QT
        pltpu.sync_copy(u_hbm, u_v)
        pltpu.sync_copy(v_hbm, v_v)
        pltpu.sync_copy(i0_hbm.at[pl.ds(base, QT)], i0_v)
        pltpu.sync_copy(i1_hbm.at[pl.ds(base, QT)], i1_v)

        def step(i, carry):
            for j in range(KD // 16):
                a = plsc.load_gather(u_v, [i0_v[pl.ds(i * KD + j * 16, 16)]])
                b = plsc.load_gather(v_v, [i1_v[pl.ds(i * KD + j * 16, 16)]])
                x = a + b
                r_v[pl.ds(j * 16, 16)] = 1.0 / (1.0 + jnp.exp(-x))
            off = pl.multiple_of(base + i * KD, 8)
            pltpu.sync_copy(r_v, out_hbm.at[pl.ds(off, KD)])
            return carry

        lax.fori_loop(0, DTRIPS, step, 0)

    return pl.kernel(body, out_type=out_type, mesh=mesh, scratch_types=scratch,
                     compiler_params=pltpu.CompilerParams(
                         use_tc_tiling_on_sc=False, needs_layout_passes=False))


_decode_cache = functools.lru_cache(maxsize=None)(_make_decode)


def kernel(x0, x1, edge_index, index, lin0_W, lin0_b, lin1_W, lin1_b,
           W1_rel, W1_root, b1, W2_rel, W2_root, b2, fc_W, fc_b):
    src = jnp.asarray(edge_index[0], _i32)
    dst = jnp.asarray(edge_index[1], _i32)
    i0 = jnp.asarray(index[0], _i32)
    i1 = jnp.asarray(index[1], _i32)

    i0p = jnp.concatenate([i0, jnp.zeros((QP - Q,), _i32)])
    i1p = jnp.concatenate([i1, jnp.zeros((QP - Q,), _i32)])

    t1_0, t1_1, root1 = _m1(
        x0, x1, lin0_W, lin0_b.reshape(1, IN), lin1_W, lin1_b.reshape(1, IN),
        W1_rel, W1_root, b1.reshape(1, HID))
    agg1_0, agg1_1, cnt0, cnt1 = _agg_cache(HID, True)(t1_0, t1_1, src, dst)
    t2p, root2 = _m2(
        root1, agg1_0, agg1_1, cnt0, cnt1, W2_rel, W2_root, b2.reshape(1, OUT))
    agg2_0, agg2_1 = _agg_cache(2 * OUT, False)(t2p, t2p, src, dst)

    wuv = jnp.concatenate([fc_W[:OUT], fc_W[OUT:]], axis=1)
    buv = jnp.concatenate([fc_b, jnp.zeros((1,), _f32)]).reshape(1, 2)
    uv = _m3(root2, agg2_0, agg2_1, cnt0, cnt1, wuv, buv)
    out = _decode_cache()(uv[:, 0], uv[:, 1], i0p, i1p)
    return out[:Q].reshape(Q, 1)


# speedup vs baseline: 2.6320x; 1.3392x over previous
import functools

import jax
import jax.numpy as jnp
from jax import lax
from jax.experimental import pallas as pl
from jax.experimental.pallas import tpu as pltpu
from jax.experimental.pallas import tpu_sc as plsc

N0 = 5000
N1 = 5000
N = N0 + N1
E = 320000
ER = E // 2
Q = 100000
IN = 128
HID = 128
OUT = 64

NC = 2
NS = 16
NW = NC * NS
K = 80
CE = ER // NS
TRIPS = CE // K
ZROWS = 80
CHUNK = 640
TAIL = N - (NS - 1) * CHUNK
CNT_CHUNK = 640
QP = 102400
QT = QP // NW
KD = 128
DTRIPS = QT // KD

_f32 = jnp.float32
_i32 = jnp.int32


def _m1_body(x0_ref, x1_ref, lw0_ref, lb0_ref, lw1_ref, lb1_ref, wrel_ref,
             wroot_ref, b1_ref, t0_ref, t1_ref, root_ref):
    h0 = jnp.dot(x0_ref[...], lw0_ref[...], preferred_element_type=_f32) + lb0_ref[...]
    h1 = jnp.dot(x1_ref[...], lw1_ref[...], preferred_element_type=_f32) + lb1_ref[...]
    h = jnp.concatenate([h0, h1], axis=0)
    t0_ref[...] = jnp.dot(h, wrel_ref[0], preferred_element_type=_f32)
    t1_ref[...] = jnp.dot(h, wrel_ref[1], preferred_element_type=_f32)
    root_ref[...] = jnp.dot(h, wroot_ref[...], preferred_element_type=_f32) + b1_ref[...]


_m1 = pl.pallas_call(
    _m1_body,
    out_shape=[
        jax.ShapeDtypeStruct((N, HID), _f32),
        jax.ShapeDtypeStruct((N, HID), _f32),
        jax.ShapeDtypeStruct((N, HID), _f32),
    ],
)


def _m2_body(root_ref, a0_ref, a1_ref, c0_ref, c1_ref, wrel_ref, wroot_ref,
             b_ref, tp_ref, root2_ref):
    inv0 = 1.0 / jnp.maximum(c0_ref[...], 1.0)
    inv1 = 1.0 / jnp.maximum(c1_ref[...], 1.0)
    z = root_ref[...] + a0_ref[...] * inv0[:, None] + a1_ref[...] * inv1[:, None]
    z = jnp.maximum(z, 0.0)
    tp_ref[...] = jnp.concatenate(
        [jnp.dot(z, wrel_ref[0], preferred_element_type=_f32),
         jnp.dot(z, wrel_ref[1], preferred_element_type=_f32)], axis=1)
    root2_ref[...] = jnp.dot(z, wroot_ref[...], preferred_element_type=_f32) + b_ref[...]


_m2 = pl.pallas_call(
    _m2_body,
    out_shape=[
        jax.ShapeDtypeStruct((N, 2 * OUT), _f32),
        jax.ShapeDtypeStruct((N, OUT), _f32),
    ],
)


def _m3_body(root_ref, a0_ref, a1_ref, c0_ref, c1_ref, wuv_ref, buv_ref, uv_ref):
    inv0 = 1.0 / jnp.maximum(c0_ref[...], 1.0)
    inv1 = 1.0 / jnp.maximum(c1_ref[...], 1.0)
    a0 = a0_ref[...][:, :OUT]
    a1 = a1_ref[...][:, OUT:]
    z = root_ref[...] + a0 * inv0[:, None] + a1 * inv1[:, None]
    uv_ref[...] = jnp.dot(z, wuv_ref[...], preferred_element_type=_f32) + buv_ref[...]


_m3 = pl.pallas_call(
    _m3_body,
    out_shape=jax.ShapeDtypeStruct((N, 2), _f32),
)


def _zero_rows(ref, rows, d):
    def row_body(r, carry):
        def col_body(j, carry2):
            ref[r, pl.ds(j * 16, 16)] = jnp.zeros((16,), _f32)
            return carry2
        return lax.fori_loop(0, d // 16, col_body, carry)
    lax.fori_loop(0, rows, row_body, 0)


def _fill_vec(ref, n, value):
    def body(j, carry):
        ref[pl.ds(j * 16, 16)] = jnp.full((16,), value, _f32)
        return carry
    lax.fori_loop(0, n // 16, body, 0)


def _make_agg(d, with_counts):
    mesh = plsc.VectorSubcoreMesh(
        core_axis_name="c", subcore_axis_name="s", num_cores=NC, num_subcores=NS)
    out_type = [
        jax.ShapeDtypeStruct((N, d), _f32),
        jax.ShapeDtypeStruct((N, d), _f32),
    ]
    scratch = [
        pltpu.VMEM((K,), _i32), pltpu.VMEM((K,), _i32),
        pltpu.VMEM((K,), _i32), pltpu.VMEM((K,), _i32),
        pltpu.VMEM((K, d), _f32), pltpu.VMEM((K, d), _f32),
        pltpu.VMEM((ZROWS, d), _f32),
        pltpu.VMEM_SHARED((N, d), _f32),
        pltpu.SemaphoreType.DMA, pltpu.SemaphoreType.DMA,
        pltpu.SemaphoreType.DMA, pltpu.SemaphoreType.DMA,
    ]
    if with_counts:
        out_type += [
            jax.ShapeDtypeStruct((N,), _f32),
            jax.ShapeDtypeStruct((N,), _f32),
        ]
        scratch += [
            pltpu.VMEM((K,), _f32),
            pltpu.VMEM((CNT_CHUNK,), _f32),
            pltpu.VMEM_SHARED((N,), _f32),
            pltpu.SemaphoreType.DMA, pltpu.SemaphoreType.DMA,
        ]

    def body(t0_hbm, t1_hbm, src_hbm, dst_hbm, agg0_out, agg1_out, *rest):
        if with_counts:
            (cnt0_out, cnt1_out, sa_v, sb_v, da_v, db_v, ra_v, rb_v,
             zrows_v, acc_sh, ssemA, ssemB, isemA, isemB,
             ones_v, zcnt_v, cnt_sh, csemA, csemB) = rest
        else:
            (sa_v, sb_v, da_v, db_v, ra_v, rb_v,
             zrows_v, acc_sh, ssemA, ssemB, isemA, isemB) = rest
        c = lax.axis_index("c")
        s = lax.axis_index("s")
        base_e = c * ER + s * CE

        _zero_rows(zrows_v, ZROWS, d)

        @pl.when(s < NS - 1)
        def _():
            for kk in range(CHUNK // ZROWS):
                pltpu.sync_copy(zrows_v, acc_sh.at[pl.ds(s * CHUNK + kk * ZROWS, ZROWS)])

        @pl.when(s == NS - 1)
        def _():
            for kk in range(TAIL // ZROWS):
                pltpu.sync_copy(zrows_v, acc_sh.at[pl.ds((NS - 1) * CHUNK + kk * ZROWS, ZROWS)])

        if with_counts:
            _fill_vec(ones_v, K, 1.0)
            _fill_vec(zcnt_v, CNT_CHUNK, 0.0)

            @pl.when(s < NS - 1)
            def _():
                pltpu.sync_copy(zcnt_v, cnt_sh.at[pl.ds(s * CNT_CHUNK, CNT_CHUNK)])

            @pl.when(s == NS - 1)
            def _():
                pltpu.sync_copy(zcnt_v.at[pl.ds(0, N - (NS - 1) * CNT_CHUNK)],
                                cnt_sh.at[pl.ds((NS - 1) * CNT_CHUNK,
                                                N - (NS - 1) * CNT_CHUNK)])
        plsc.subcore_barrier()

        def issue_idx(t, sidx_v, didx_v, sem):
            off = pl.multiple_of(base_e + t * K, 8)
            pltpu.async_copy(src_hbm.at[pl.ds(off, K)], sidx_v, sem)
            pltpu.async_copy(dst_hbm.at[pl.ds(off, K)], didx_v, sem)

        def drain_idx(sidx_v, didx_v, sem):
            pltpu.make_async_copy(src_hbm.at[pl.ds(0, K)], sidx_v, sem).wait()
            pltpu.make_async_copy(src_hbm.at[pl.ds(0, K)], didx_v, sem).wait()

        def gather(sidx_v, rows_v):
            @pl.when(c == 0)
            def _():
                pltpu.sync_copy(t0_hbm.at[sidx_v], rows_v)

            @pl.when(c == 1)
            def _():
                pltpu.sync_copy(t1_hbm.at[sidx_v], rows_v)

        def issue_scatter(didx_v, rows_v, ssem, csem):
            descs = [pltpu.async_copy(rows_v, acc_sh.at[didx_v], ssem, add=True)]
            if with_counts:
                descs.append(
                    pltpu.async_copy(ones_v, cnt_sh.at[didx_v], csem, add=True))
            return descs

        def drain_b():
            pltpu.make_async_copy(t0_hbm.at[pl.ds(0, K)], rb_v, ssemB).wait()
            if with_counts:
                pltpu.make_async_copy(cnt0_out.at[pl.ds(0, K)], ones_v, csemB).wait()

        issue_idx(0, sa_v, da_v, isemA)

        def group(g, carry):
            issue_idx(2 * g + 1, sb_v, db_v, isemB)
            drain_idx(sa_v, da_v, isemA)
            gather(sa_v, ra_v)

            @pl.when(g > 0)
            def _():
                drain_b()

            da = issue_scatter(da_v, ra_v, ssemA, csemA if with_counts else None)
            drain_idx(sb_v, db_v, isemB)
            gather(sb_v, rb_v)
            for dsc in da:
                dsc.wait()
            issue_idx(2 * g + 2, sa_v, da_v, isemA)
            issue_scatter(db_v, rb_v, ssemB, csemB if with_counts else None)
            return carry

        lax.fori_loop(0, TRIPS // 2, group, 0)
        drain_b()
        drain_idx(sa_v, da_v, isemA)
        if TRIPS % 2:
            gather(sa_v, ra_v)
            for dsc in issue_scatter(da_v, ra_v, ssemA, csemA if with_counts else None):
                dsc.wait()
        plsc.subcore_barrier()

        for cc, agg_out in ((0, agg0_out), (1, agg1_out)):
            @pl.when(jnp.logical_and(c == cc, s < NS - 1))
            def _(agg_out=agg_out):
                pltpu.sync_copy(acc_sh.at[pl.ds(s * CHUNK, CHUNK)],
                                agg_out.at[pl.ds(s * CHUNK, CHUNK)])

            @pl.when(jnp.logical_and(c == cc, s == NS - 1))
            def _(agg_out=agg_out):
                pltpu.sync_copy(acc_sh.at[pl.ds((NS - 1) * CHUNK, TAIL)],
                                agg_out.at[pl.ds((NS - 1) * CHUNK, TAIL)])

        if with_counts:
            tail = N - (NS - 1) * CNT_CHUNK
            for cc, cnt_out in ((0, cnt0_out), (1, cnt1_out)):
                @pl.when(jnp.logical_and(c == cc, s < NS - 1))
                def _(cnt_out=cnt_out):
                    pltpu.sync_copy(cnt_sh.at[pl.ds(s * CNT_CHUNK, CNT_CHUNK)], zcnt_v)
                    pltpu.sync_copy(zcnt_v, cnt_out.at[pl.ds(s * CNT_CHUNK, CNT_CHUNK)])

                @pl.when(jnp.logical_and(c == cc, s == NS - 1))
                def _(cnt_out=cnt_out):
                    pltpu.sync_copy(cnt_sh.at[pl.ds((NS - 1) * CNT_CHUNK, tail)],
                                    zcnt_v.at[pl.ds(0, tail)])
                    pltpu.sync_copy(zcnt_v.at[pl.ds(0, tail)],
                                    cnt_out.at[pl.ds((NS - 1) * CNT_CHUNK, tail)])

    return pl.kernel(body, out_type=out_type, mesh=mesh, scratch_types=scratch)


_agg_cache = functools.lru_cache(maxsize=None)(_make_agg)


def _make_decode():
    mesh = plsc.VectorSubcoreMesh(
        core_axis_name="c", subcore_axis_name="s", num_cores=NC, num_subcores=NS)
    out_type = jax.ShapeDtypeStruct((QP,), _f32)
    scratch = [
        pltpu.VMEM((N,), _f32),
        pltpu.VMEM((N,), _f32),
        pltpu.VMEM((QT,), _i32),
        pltpu.VMEM((QT,), _i32),
        pltpu.VMEM((KD,), _f32),
    ]

    def body(u_hbm, v_hbm, i0_hbm, i1_hbm, out_hbm, u_v, v_v, i0_v, i1_v, r_v):
        c = lax.axis_index("c")
        s = lax.axis_index("s")
        w = s * NC + c
        base = w * QT
        pltpu.sync_copy(u_hbm, u_v)
        pltpu.sync_copy(v_hbm, v_v)
        pltpu.sync_copy(i0_hbm.at[pl.ds(base, QT)], i0_v)
        pltpu.sync_copy(i1_hbm.at[pl.ds(base, QT)], i1_v)

        def step(i, carry):
            for j in range(KD // 16):
                a = plsc.load_gather(u_v, [i0_v[pl.ds(i * KD + j * 16, 16)]])
                b = plsc.load_gather(v_v, [i1_v[pl.ds(i * KD + j * 16, 16)]])
                x = a + b
                r_v[pl.ds(j * 16, 16)] = 1.0 / (1.0 + jnp.exp(-x))
            off = pl.multiple_of(base + i * KD, 8)
            pltpu.sync_copy(r_v, out_hbm.at[pl.ds(off, KD)])
            return carry

        lax.fori_loop(0, DTRIPS, step, 0)

    return pl.kernel(body, out_type=out_type, mesh=mesh, scratch_types=scratch,
                     compiler_params=pltpu.CompilerParams(
                         use_tc_tiling_on_sc=False, needs_layout_passes=False))


_decode_cache = functools.lru_cache(maxsize=None)(_make_decode)


def kernel(x0, x1, edge_index, index, lin0_W, lin0_b, lin1_W, lin1_b,
           W1_rel, W1_root, b1, W2_rel, W2_root, b2, fc_W, fc_b):
    src = jnp.asarray(edge_index[0], _i32)
    dst = jnp.asarray(edge_index[1], _i32)
    i0 = jnp.asarray(index[0], _i32)
    i1 = jnp.asarray(index[1], _i32)

    i0p = jnp.concatenate([i0, jnp.zeros((QP - Q,), _i32)])
    i1p = jnp.concatenate([i1, jnp.zeros((QP - Q,), _i32)])

    t1_0, t1_1, root1 = _m1(
        x0, x1, lin0_W, lin0_b.reshape(1, IN), lin1_W, lin1_b.reshape(1, IN),
        W1_rel, W1_root, b1.reshape(1, HID))
    agg1_0, agg1_1, cnt0, cnt1 = _agg_cache(HID, True)(t1_0, t1_1, src, dst)
    t2p, root2 = _m2(
        root1, agg1_0, agg1_1, cnt0, cnt1, W2_rel, W2_root, b2.reshape(1, OUT))
    agg2_0, agg2_1 = _agg_cache(2 * OUT, False)(t2p, t2p, src, dst)

    wuv = jnp.concatenate([fc_W[:OUT], fc_W[OUT:]], axis=1)
    buv = jnp.concatenate([fc_b, jnp.zeros((1,), _f32)]).reshape(1, 2)
    uv = _m3(root2, agg2_0, agg2_1, cnt0, cnt1, wuv, buv)
    out = _decode_cache()(uv[:, 0], uv[:, 1], i0p, i1p)
    return out[:Q].reshape(Q, 1)


# speedup vs baseline: 2.8375x; 1.0781x over previous
import functools

import jax
import jax.numpy as jnp
from jax import lax
from jax.experimental import pallas as pl
from jax.experimental.pallas import tpu as pltpu
from jax.experimental.pallas import tpu_sc as plsc

N0 = 5000
N1 = 5000
N = N0 + N1
E = 320000
ER = E // 2
Q = 100000
IN = 128
HID = 128
OUT = 64

NC = 2
NS = 16
NW = NC * NS
K = 80
CE = ER // NS
TRIPS = CE // K
ZROWS = 80
CHUNK = 640
TAIL = N - (NS - 1) * CHUNK
CNT_CHUNK = 640
QP = 102400
QT = QP // NW
KD = 128
DTRIPS = QT // KD

_f32 = jnp.float32
_i32 = jnp.int32


def _m1_body(x0_ref, x1_ref, lw0_ref, lb0_ref, lw1_ref, lb1_ref, wrel_ref,
             wroot_ref, b1_ref, t0_ref, t1_ref, root_ref):
    h0 = jnp.dot(x0_ref[...], lw0_ref[...], preferred_element_type=_f32) + lb0_ref[...]
    h1 = jnp.dot(x1_ref[...], lw1_ref[...], preferred_element_type=_f32) + lb1_ref[...]
    h = jnp.concatenate([h0, h1], axis=0)
    t0_ref[...] = jnp.dot(h, wrel_ref[0], preferred_element_type=_f32)
    t1_ref[...] = jnp.dot(h, wrel_ref[1], preferred_element_type=_f32)
    root_ref[...] = jnp.dot(h, wroot_ref[...], preferred_element_type=_f32) + b1_ref[...]


_m1 = pl.pallas_call(
    _m1_body,
    out_shape=[
        jax.ShapeDtypeStruct((N, HID), _f32),
        jax.ShapeDtypeStruct((N, HID), _f32),
        jax.ShapeDtypeStruct((N, HID), _f32),
    ],
)


def _m2_body(root_ref, a0_ref, a1_ref, c0_ref, c1_ref, wrel_ref, wroot_ref,
             b_ref, tp_ref, root2_ref):
    inv0 = 1.0 / jnp.maximum(c0_ref[...], 1.0)
    inv1 = 1.0 / jnp.maximum(c1_ref[...], 1.0)
    z = root_ref[...] + a0_ref[...] * inv0[:, None] + a1_ref[...] * inv1[:, None]
    z = jnp.maximum(z, 0.0)
    tp_ref[...] = jnp.concatenate(
        [jnp.dot(z, wrel_ref[0], preferred_element_type=_f32),
         jnp.dot(z, wrel_ref[1], preferred_element_type=_f32)], axis=1)
    root2_ref[...] = jnp.dot(z, wroot_ref[...], preferred_element_type=_f32) + b_ref[...]


_m2 = pl.pallas_call(
    _m2_body,
    out_shape=[
        jax.ShapeDtypeStruct((N, 2 * OUT), _f32),
        jax.ShapeDtypeStruct((N, OUT), _f32),
    ],
)


def _m3_body(root_ref, a0_ref, a1_ref, c0_ref, c1_ref, wuv_ref, buv_ref, uv_ref):
    inv0 = 1.0 / jnp.maximum(c0_ref[...], 1.0)
    inv1 = 1.0 / jnp.maximum(c1_ref[...], 1.0)
    a0 = a0_ref[...][:, :OUT]
    a1 = a1_ref[...][:, OUT:]
    z = root_ref[...] + a0 * inv0[:, None] + a1 * inv1[:, None]
    uv_ref[...] = jnp.dot(z, wuv_ref[...], preferred_element_type=_f32) + buv_ref[...]


_m3 = pl.pallas_call(
    _m3_body,
    out_shape=jax.ShapeDtypeStruct((N, 2), _f32),
)


def _zero_rows(ref, rows, d):
    def row_body(r, carry):
        def col_body(j, carry2):
            ref[r, pl.ds(j * 16, 16)] = jnp.zeros((16,), _f32)
            return carry2
        return lax.fori_loop(0, d // 16, col_body, carry)
    lax.fori_loop(0, rows, row_body, 0)


def _fill_vec(ref, n, value):
    def body(j, carry):
        ref[pl.ds(j * 16, 16)] = jnp.full((16,), value, _f32)
        return carry
    lax.fori_loop(0, n // 16, body, 0)


def _make_agg(d, with_counts):
    mesh = plsc.VectorSubcoreMesh(
        core_axis_name="c", subcore_axis_name="s", num_cores=NC, num_subcores=NS)
    out_type = [
        jax.ShapeDtypeStruct((N, d), _f32),
        jax.ShapeDtypeStruct((N, d), _f32),
    ]
    scratch = [
        pltpu.VMEM((K,), _i32), pltpu.VMEM((K,), _i32),
        pltpu.VMEM((K,), _i32), pltpu.VMEM((K,), _i32),
        pltpu.VMEM((K, d), _f32), pltpu.VMEM((K, d), _f32),
        pltpu.VMEM((ZROWS, d), _f32),
        pltpu.VMEM_SHARED((N, d), _f32),
        pltpu.SemaphoreType.DMA, pltpu.SemaphoreType.DMA,
        pltpu.SemaphoreType.DMA, pltpu.SemaphoreType.DMA,
        pltpu.SemaphoreType.DMA, pltpu.SemaphoreType.DMA,
    ]
    if with_counts:
        out_type += [
            jax.ShapeDtypeStruct((N,), _f32),
            jax.ShapeDtypeStruct((N,), _f32),
        ]
        scratch += [
            pltpu.VMEM((K,), _f32),
            pltpu.VMEM((CNT_CHUNK,), _f32),
            pltpu.VMEM_SHARED((N,), _f32),
            pltpu.SemaphoreType.DMA, pltpu.SemaphoreType.DMA,
        ]

    def body(t0_hbm, t1_hbm, src_hbm, dst_hbm, agg0_out, agg1_out, *rest):
        if with_counts:
            (cnt0_out, cnt1_out, sa_v, sb_v, da_v, db_v, ra_v, rb_v,
             zrows_v, acc_sh, ssemA, ssemB, isemA, isemB, gsemA, gsemB,
             ones_v, zcnt_v, cnt_sh, csemA, csemB) = rest
        else:
            (sa_v, sb_v, da_v, db_v, ra_v, rb_v,
             zrows_v, acc_sh, ssemA, ssemB, isemA, isemB, gsemA, gsemB) = rest
        c = lax.axis_index("c")
        s = lax.axis_index("s")
        base_e = c * ER + s * CE

        _zero_rows(zrows_v, ZROWS, d)

        @pl.when(s < NS - 1)
        def _():
            for kk in range(CHUNK // ZROWS):
                pltpu.sync_copy(zrows_v, acc_sh.at[pl.ds(s * CHUNK + kk * ZROWS, ZROWS)])

        @pl.when(s == NS - 1)
        def _():
            for kk in range(TAIL // ZROWS):
                pltpu.sync_copy(zrows_v, acc_sh.at[pl.ds((NS - 1) * CHUNK + kk * ZROWS, ZROWS)])

        if with_counts:
            _fill_vec(ones_v, K, 1.0)
            _fill_vec(zcnt_v, CNT_CHUNK, 0.0)

            @pl.when(s < NS - 1)
            def _():
                pltpu.sync_copy(zcnt_v, cnt_sh.at[pl.ds(s * CNT_CHUNK, CNT_CHUNK)])

            @pl.when(s == NS - 1)
            def _():
                pltpu.sync_copy(zcnt_v.at[pl.ds(0, N - (NS - 1) * CNT_CHUNK)],
                                cnt_sh.at[pl.ds((NS - 1) * CNT_CHUNK,
                                                N - (NS - 1) * CNT_CHUNK)])
        plsc.subcore_barrier()

        def issue_idx(t, sidx_v, didx_v, sem):
            off = pl.multiple_of(base_e + t * K, 8)
            pltpu.async_copy(src_hbm.at[pl.ds(off, K)], sidx_v, sem)
            pltpu.async_copy(dst_hbm.at[pl.ds(off, K)], didx_v, sem)

        def drain_idx(sidx_v, didx_v, sem):
            pltpu.make_async_copy(src_hbm.at[pl.ds(0, K)], sidx_v, sem).wait()
            pltpu.make_async_copy(src_hbm.at[pl.ds(0, K)], didx_v, sem).wait()

        def gather(sidx_v, rows_v):
            @pl.when(c == 0)
            def _():
                pltpu.sync_copy(t0_hbm.at[sidx_v], rows_v)

            @pl.when(c == 1)
            def _():
                pltpu.sync_copy(t1_hbm.at[sidx_v], rows_v)

        def issue_gather(sidx_v, rows_v, gsem):
            @pl.when(c == 0)
            def _():
                pltpu.async_copy(t0_hbm.at[sidx_v], rows_v, gsem)

            @pl.when(c == 1)
            def _():
                pltpu.async_copy(t1_hbm.at[sidx_v], rows_v, gsem)

        def drain_gather(rows_v, gsem):
            pltpu.make_async_copy(t0_hbm.at[pl.ds(0, K)], rows_v, gsem).wait()

        def issue_scatter(didx_v, rows_v, ssem, csem):
            descs = [pltpu.async_copy(rows_v, acc_sh.at[didx_v], ssem, add=True)]
            if with_counts:
                descs.append(
                    pltpu.async_copy(ones_v, cnt_sh.at[didx_v], csem, add=True))
            return descs

        def drain_b():
            pltpu.make_async_copy(t0_hbm.at[pl.ds(0, K)], rb_v, ssemB).wait()
            if with_counts:
                pltpu.make_async_copy(cnt0_out.at[pl.ds(0, K)], ones_v, csemB).wait()

        issue_idx(0, sa_v, da_v, isemA)

        def group(g, carry):
            issue_idx(2 * g + 1, sb_v, db_v, isemB)
            drain_idx(sa_v, da_v, isemA)
            issue_gather(sa_v, ra_v, gsemA)
            drain_idx(sb_v, db_v, isemB)
            issue_gather(sb_v, rb_v, gsemB)

            @pl.when(g > 0)
            def _():
                drain_b()

            drain_gather(ra_v, gsemA)
            da = issue_scatter(da_v, ra_v, ssemA, csemA if with_counts else None)
            drain_gather(rb_v, gsemB)
            for dsc in da:
                dsc.wait()
            issue_idx(2 * g + 2, sa_v, da_v, isemA)
            issue_scatter(db_v, rb_v, ssemB, csemB if with_counts else None)
            return carry

        lax.fori_loop(0, TRIPS // 2, group, 0)
        drain_b()
        drain_idx(sa_v, da_v, isemA)
        if TRIPS % 2:
            gather(sa_v, ra_v)
            for dsc in issue_scatter(da_v, ra_v, ssemA, csemA if with_counts else None):
                dsc.wait()
        plsc.subcore_barrier()

        for cc, agg_out in ((0, agg0_out), (1, agg1_out)):
            @pl.when(jnp.logical_and(c == cc, s < NS - 1))
            def _(agg_out=agg_out):
                pltpu.sync_copy(acc_sh.at[pl.ds(s * CHUNK, CHUNK)],
                                agg_out.at[pl.ds(s * CHUNK, CHUNK)])

            @pl.when(jnp.logical_and(c == cc, s == NS - 1))
            def _(agg_out=agg_out):
                pltpu.sync_copy(acc_sh.at[pl.ds((NS - 1) * CHUNK, TAIL)],
                                agg_out.at[pl.ds((NS - 1) * CHUNK, TAIL)])

        if with_counts:
            tail = N - (NS - 1) * CNT_CHUNK
            for cc, cnt_out in ((0, cnt0_out), (1, cnt1_out)):
                @pl.when(jnp.logical_and(c == cc, s < NS - 1))
                def _(cnt_out=cnt_out):
                    pltpu.sync_copy(cnt_sh.at[pl.ds(s * CNT_CHUNK, CNT_CHUNK)], zcnt_v)
                    pltpu.sync_copy(zcnt_v, cnt_out.at[pl.ds(s * CNT_CHUNK, CNT_CHUNK)])

                @pl.when(jnp.logical_and(c == cc, s == NS - 1))
                def _(cnt_out=cnt_out):
                    pltpu.sync_copy(cnt_sh.at[pl.ds((NS - 1) * CNT_CHUNK, tail)],
                                    zcnt_v.at[pl.ds(0, tail)])
                    pltpu.sync_copy(zcnt_v.at[pl.ds(0, tail)],
                                    cnt_out.at[pl.ds((NS - 1) * CNT_CHUNK, tail)])

    return pl.kernel(body, out_type=out_type, mesh=mesh, scratch_types=scratch)


_agg_cache = functools.lru_cache(maxsize=None)(_make_agg)


def _make_decode():
    mesh = plsc.VectorSubcoreMesh(
        core_axis_name="c", subcore_axis_name="s", num_cores=NC, num_subcores=NS)
    out_type = jax.ShapeDtypeStruct((QP,), _f32)
    scratch = [
        pltpu.VMEM((N,), _f32),
        pltpu.VMEM((N,), _f32),
        pltpu.VMEM((QT,), _i32),
        pltpu.VMEM((QT,), _i32),
        pltpu.VMEM((KD,), _f32),
    ]

    def body(u_hbm, v_hbm, i0_hbm, i1_hbm, out_hbm, u_v, v_v, i0_v, i1_v, r_v):
        c = lax.axis_index("c")
        s = lax.axis_index("s")
        w = s * NC + c
        base = w * QT
        pltpu.sync_copy(u_hbm, u_v)
        pltpu.sync_copy(v_hbm, v_v)
        pltpu.sync_copy(i0_hbm.at[pl.ds(base, QT)], i0_v)
        pltpu.sync_copy(i1_hbm.at[pl.ds(base, QT)], i1_v)

        def step(i, carry):
            for j in range(KD // 16):
                a = plsc.load_gather(u_v, [i0_v[pl.ds(i * KD + j * 16, 16)]])
                b = plsc.load_gather(v_v, [i1_v[pl.ds(i * KD + j * 16, 16)]])
                x = a + b
                r_v[pl.ds(j * 16, 16)] = 1.0 / (1.0 + jnp.exp(-x))
            off = pl.multiple_of(base + i * KD, 8)
            pltpu.sync_copy(r_v, out_hbm.at[pl.ds(off, KD)])
            return carry

        lax.fori_loop(0, DTRIPS, step, 0)

    return pl.kernel(body, out_type=out_type, mesh=mesh, scratch_types=scratch,
                     compiler_params=pltpu.CompilerParams(
                         use_tc_tiling_on_sc=False, needs_layout_passes=False))


_decode_cache = functools.lru_cache(maxsize=None)(_make_decode)


def kernel(x0, x1, edge_index, index, lin0_W, lin0_b, lin1_W, lin1_b,
           W1_rel, W1_root, b1, W2_rel, W2_root, b2, fc_W, fc_b):
    src = jnp.asarray(edge_index[0], _i32)
    dst = jnp.asarray(edge_index[1], _i32)
    i0 = jnp.asarray(index[0], _i32)
    i1 = jnp.asarray(index[1], _i32)

    i0p = jnp.concatenate([i0, jnp.zeros((QP - Q,), _i32)])
    i1p = jnp.concatenate([i1, jnp.zeros((QP - Q,), _i32)])

    t1_0, t1_1, root1 = _m1(
        x0, x1, lin0_W, lin0_b.reshape(1, IN), lin1_W, lin1_b.reshape(1, IN),
        W1_rel, W1_root, b1.reshape(1, HID))
    agg1_0, agg1_1, cnt0, cnt1 = _agg_cache(HID, True)(t1_0, t1_1, src, dst)
    t2p, root2 = _m2(
        root1, agg1_0, agg1_1, cnt0, cnt1, W2_rel, W2_root, b2.reshape(1, OUT))
    agg2_0, agg2_1 = _agg_cache(2 * OUT, False)(t2p, t2p, src, dst)

    wuv = jnp.concatenate([fc_W[:OUT], fc_W[OUT:]], axis=1)
    buv = jnp.concatenate([fc_b, jnp.zeros((1,), _f32)]).reshape(1, 2)
    uv = _m3(root2, agg2_0, agg2_1, cnt0, cnt1, wuv, buv)
    out = _decode_cache()(uv[:, 0], uv[:, 1], i0p, i1p)
    return out[:Q].reshape(Q, 1)


# speedup vs baseline: 3.6812x; 1.2973x over previous
import functools

import jax
import jax.numpy as jnp
from jax import lax
from jax.experimental import pallas as pl
from jax.experimental.pallas import tpu as pltpu
from jax.experimental.pallas import tpu_sc as plsc

N0 = 5000
N1 = 5000
N = N0 + N1
E = 320000
ER = E // 2
Q = 100000
IN = 128
HID = 128
OUT = 64

NC = 2
NS = 16
NW = NC * NS
K = 80
CE = ER // NS
TRIPS = CE // K
ZROWS = 80
CHUNK = 640
TAIL = N - (NS - 1) * CHUNK
CNT_CHUNK = 640
QP = 102400
QT = QP // NW
KD = 128
DTRIPS = QT // KD

_f32 = jnp.float32
_i32 = jnp.int32


def _m1_body(x0_ref, x1_ref, lw0_ref, lb0_ref, lw1_ref, lb1_ref, wrel_ref,
             wroot_ref, b1_ref, t0_ref, t1_ref, root_ref):
    h0 = jnp.dot(x0_ref[...], lw0_ref[...], preferred_element_type=_f32) + lb0_ref[...]
    h1 = jnp.dot(x1_ref[...], lw1_ref[...], preferred_element_type=_f32) + lb1_ref[...]
    h = jnp.concatenate([h0, h1], axis=0)
    t0_ref[...] = jnp.dot(h, wrel_ref[0], preferred_element_type=_f32)
    t1_ref[...] = jnp.dot(h, wrel_ref[1], preferred_element_type=_f32)
    root_ref[...] = jnp.dot(h, wroot_ref[...], preferred_element_type=_f32) + b1_ref[...]


_m1 = pl.pallas_call(
    _m1_body,
    out_shape=[
        jax.ShapeDtypeStruct((N, HID), _f32),
        jax.ShapeDtypeStruct((N, HID), _f32),
        jax.ShapeDtypeStruct((N, HID), _f32),
    ],
)


def _m2_body(root_ref, a0_ref, a1_ref, c0_ref, c1_ref, wrel_ref, wroot_ref,
             b_ref, tp_ref, root2_ref):
    inv0 = 1.0 / jnp.maximum(c0_ref[...], 1.0)
    inv1 = 1.0 / jnp.maximum(c1_ref[...], 1.0)
    z = root_ref[...] + a0_ref[...] * inv0[:, None] + a1_ref[...] * inv1[:, None]
    z = jnp.maximum(z, 0.0)
    tp_ref[...] = jnp.concatenate(
        [jnp.dot(z, wrel_ref[0], preferred_element_type=_f32),
         jnp.dot(z, wrel_ref[1], preferred_element_type=_f32)], axis=1)
    root2_ref[...] = jnp.dot(z, wroot_ref[...], preferred_element_type=_f32) + b_ref[...]


_m2 = pl.pallas_call(
    _m2_body,
    out_shape=[
        jax.ShapeDtypeStruct((N, 2 * OUT), _f32),
        jax.ShapeDtypeStruct((N, OUT), _f32),
    ],
)


def _m3_body(root_ref, a0_ref, a1_ref, c0_ref, c1_ref, wuv_ref, buv_ref, uv_ref):
    inv0 = 1.0 / jnp.maximum(c0_ref[...], 1.0)
    inv1 = 1.0 / jnp.maximum(c1_ref[...], 1.0)
    a0 = a0_ref[...][:, :OUT]
    a1 = a1_ref[...][:, OUT:]
    z = root_ref[...] + a0 * inv0[:, None] + a1 * inv1[:, None]
    uv_ref[...] = jnp.dot(z, wuv_ref[...], preferred_element_type=_f32) + buv_ref[...]


_m3 = pl.pallas_call(
    _m3_body,
    out_shape=jax.ShapeDtypeStruct((N, 2), _f32),
)


def _zero_rows(ref, rows, d):
    def row_body(r, carry):
        def col_body(j, carry2):
            ref[r, pl.ds(j * 16, 16)] = jnp.zeros((16,), _f32)
            return carry2
        return lax.fori_loop(0, d // 16, col_body, carry)
    lax.fori_loop(0, rows, row_body, 0)


def _fill_vec(ref, n, value):
    def body(j, carry):
        ref[pl.ds(j * 16, 16)] = jnp.full((16,), value, _f32)
        return carry
    lax.fori_loop(0, n // 16, body, 0)


def _make_agg(d, with_counts):
    mesh = plsc.VectorSubcoreMesh(
        core_axis_name="c", subcore_axis_name="s", num_cores=NC, num_subcores=NS)
    out_type = [
        jax.ShapeDtypeStruct((N, d), _f32),
        jax.ShapeDtypeStruct((N, d), _f32),
    ]
    NB = 3
    scratch = (
        [pltpu.VMEM((K,), _i32) for _ in range(NB)]
        + [pltpu.VMEM((K,), _i32) for _ in range(NB)]
        + [pltpu.VMEM((K, d), _f32) for _ in range(NB)]
        + [
            pltpu.VMEM((ZROWS, d), _f32),
            pltpu.VMEM_SHARED((N, d), _f32),
        ]
        + [pltpu.SemaphoreType.DMA for _ in range(3 * NB)]
    )
    if with_counts:
        out_type += [
            jax.ShapeDtypeStruct((N,), _f32),
            jax.ShapeDtypeStruct((N,), _f32),
        ]
        scratch += (
            [pltpu.VMEM((K,), _f32),
             pltpu.VMEM((CNT_CHUNK,), _f32)]
            + [pltpu.VMEM_SHARED((N,), _f32)]
            + [pltpu.SemaphoreType.DMA for _ in range(3)]
        )

    def body(t0_hbm, t1_hbm, src_hbm, dst_hbm, agg0_out, agg1_out, *rest):
        NB = 3
        sidx = list(rest[0 + 2 * with_counts:NB + 2 * with_counts])
        didx = list(rest[NB + 2 * with_counts:2 * NB + 2 * with_counts])
        rows = list(rest[2 * NB + 2 * with_counts:3 * NB + 2 * with_counts])
        base = 3 * NB + 2 * with_counts
        zrows_v, acc_sh = rest[base], rest[base + 1]
        isem = list(rest[base + 2:base + 2 + NB])
        gsem = list(rest[base + 2 + NB:base + 2 + 2 * NB])
        ssem = list(rest[base + 2 + 2 * NB:base + 2 + 3 * NB])
        if with_counts:
            cnt0_out, cnt1_out = rest[0], rest[1]
            ones_v = rest[base + 2 + 3 * NB]
            zcnt_v = rest[base + 3 + 3 * NB]
            cnt_sh = rest[base + 4 + 3 * NB]
            csem = list(rest[base + 5 + 3 * NB:base + 8 + 3 * NB])
        c = lax.axis_index("c")
        s = lax.axis_index("s")
        base_e = c * ER + s * CE

        _zero_rows(zrows_v, ZROWS, d)

        @pl.when(s < NS - 1)
        def _():
            for kk in range(CHUNK // ZROWS):
                pltpu.sync_copy(zrows_v, acc_sh.at[pl.ds(s * CHUNK + kk * ZROWS, ZROWS)])

        @pl.when(s == NS - 1)
        def _():
            for kk in range(TAIL // ZROWS):
                pltpu.sync_copy(zrows_v, acc_sh.at[pl.ds((NS - 1) * CHUNK + kk * ZROWS, ZROWS)])

        if with_counts:
            _fill_vec(ones_v, K, 1.0)
            _fill_vec(zcnt_v, CNT_CHUNK, 0.0)

            @pl.when(s < NS - 1)
            def _():
                pltpu.sync_copy(zcnt_v, cnt_sh.at[pl.ds(s * CNT_CHUNK, CNT_CHUNK)])

            @pl.when(s == NS - 1)
            def _():
                pltpu.sync_copy(zcnt_v.at[pl.ds(0, N - (NS - 1) * CNT_CHUNK)],
                                cnt_sh.at[pl.ds((NS - 1) * CNT_CHUNK,
                                                N - (NS - 1) * CNT_CHUNK)])
        plsc.subcore_barrier()

        def issue_idx(t, bi):
            off = pl.multiple_of(base_e + t * K, 8)
            pltpu.async_copy(src_hbm.at[pl.ds(off, K)], sidx[bi], isem[bi])
            pltpu.async_copy(dst_hbm.at[pl.ds(off, K)], didx[bi], isem[bi])

        def drain_idx(bi):
            pltpu.make_async_copy(src_hbm.at[pl.ds(0, K)], sidx[bi], isem[bi]).wait()
            pltpu.make_async_copy(src_hbm.at[pl.ds(0, K)], didx[bi], isem[bi]).wait()

        def issue_gather(bi):
            @pl.when(c == 0)
            def _():
                pltpu.async_copy(t0_hbm.at[sidx[bi]], rows[bi], gsem[bi])

            @pl.when(c == 1)
            def _():
                pltpu.async_copy(t1_hbm.at[sidx[bi]], rows[bi], gsem[bi])

        def drain_gather(bi):
            pltpu.make_async_copy(t0_hbm.at[pl.ds(0, K)], rows[bi], gsem[bi]).wait()

        def issue_scatter(bi):
            descs = [pltpu.async_copy(rows[bi], acc_sh.at[didx[bi]], ssem[bi],
                                      add=True)]
            if with_counts:
                descs.append(pltpu.async_copy(ones_v, cnt_sh.at[didx[bi]],
                                              csem[bi], add=True))
            return descs

        def drain_scatter(bi):
            pltpu.make_async_copy(t0_hbm.at[pl.ds(0, K)], rows[bi], ssem[bi]).wait()
            if with_counts:
                pltpu.make_async_copy(cnt0_out.at[pl.ds(0, K)], ones_v,
                                      csem[bi]).wait()

        issue_idx(0, 0)
        issue_idx(1, 1)

        def group(g, carry):
            drain_idx(0)
            issue_gather(0)

            @pl.when(g > 0)
            def _():
                drain_scatter(2)

            issue_idx(3 * g + 2, 2)
            drain_idx(1)
            issue_gather(1)
            drain_gather(0)
            d0 = issue_scatter(0)
            drain_idx(2)
            issue_gather(2)
            drain_gather(1)
            d1 = issue_scatter(1)
            for dsc in d0:
                dsc.wait()
            issue_idx(3 * g + 3, 0)
            drain_gather(2)
            for dsc in d1:
                dsc.wait()
            issue_idx(3 * g + 4, 1)
            issue_scatter(2)
            return carry

        ngroups = (TRIPS - 2) // 3
        lax.fori_loop(0, ngroups, group, 0)
        drain_scatter(2)
        for j, t in enumerate(range(3 * ngroups, TRIPS)):
            drain_idx(j)
            issue_gather(j)
            drain_gather(j)
            for dsc in issue_scatter(j):
                dsc.wait()
        plsc.subcore_barrier()

        for cc, agg_out in ((0, agg0_out), (1, agg1_out)):
            @pl.when(jnp.logical_and(c == cc, s < NS - 1))
            def _(agg_out=agg_out):
                pltpu.sync_copy(acc_sh.at[pl.ds(s * CHUNK, CHUNK)],
                                agg_out.at[pl.ds(s * CHUNK, CHUNK)])

            @pl.when(jnp.logical_and(c == cc, s == NS - 1))
            def _(agg_out=agg_out):
                pltpu.sync_copy(acc_sh.at[pl.ds((NS - 1) * CHUNK, TAIL)],
                                agg_out.at[pl.ds((NS - 1) * CHUNK, TAIL)])

        if with_counts:
            tail = N - (NS - 1) * CNT_CHUNK
            for cc, cnt_out in ((0, cnt0_out), (1, cnt1_out)):
                @pl.when(jnp.logical_and(c == cc, s < NS - 1))
                def _(cnt_out=cnt_out):
                    pltpu.sync_copy(cnt_sh.at[pl.ds(s * CNT_CHUNK, CNT_CHUNK)], zcnt_v)
                    pltpu.sync_copy(zcnt_v, cnt_out.at[pl.ds(s * CNT_CHUNK, CNT_CHUNK)])

                @pl.when(jnp.logical_and(c == cc, s == NS - 1))
                def _(cnt_out=cnt_out):
                    pltpu.sync_copy(cnt_sh.at[pl.ds((NS - 1) * CNT_CHUNK, tail)],
                                    zcnt_v.at[pl.ds(0, tail)])
                    pltpu.sync_copy(zcnt_v.at[pl.ds(0, tail)],
                                    cnt_out.at[pl.ds((NS - 1) * CNT_CHUNK, tail)])

    return pl.kernel(body, out_type=out_type, mesh=mesh, scratch_types=scratch)


_agg_cache = functools.lru_cache(maxsize=None)(_make_agg)


def _make_decode():
    mesh = plsc.VectorSubcoreMesh(
        core_axis_name="c", subcore_axis_name="s", num_cores=NC, num_subcores=NS)
    out_type = jax.ShapeDtypeStruct((QP,), _f32)
    scratch = [
        pltpu.VMEM((N,), _f32),
        pltpu.VMEM((N,), _f32),
        pltpu.VMEM((QT,), _i32),
        pltpu.VMEM((QT,), _i32),
        pltpu.VMEM((KD,), _f32),
    ]

    def body(u_hbm, v_hbm, i0_hbm, i1_hbm, out_hbm, u_v, v_v, i0_v, i1_v, r_v):
        c = lax.axis_index("c")
        s = lax.axis_index("s")
        w = s * NC + c
        base = w * QT
        pltpu.sync_copy(u_hbm, u_v)
        pltpu.sync_copy(v_hbm, v_v)
        pltpu.sync_copy(i0_hbm.at[pl.ds(base, QT)], i0_v)
        pltpu.sync_copy(i1_hbm.at[pl.ds(base, QT)], i1_v)

        def step(i, carry):
            for j in range(KD // 16):
                a = plsc.load_gather(u_v, [i0_v[pl.ds(i * KD + j * 16, 16)]])
                b = plsc.load_gather(v_v, [i1_v[pl.ds(i * KD + j * 16, 16)]])
                x = a + b
                r_v[pl.ds(j * 16, 16)] = 1.0 / (1.0 + jnp.exp(-x))
            off = pl.multiple_of(base + i * KD, 8)
            pltpu.sync_copy(r_v, out_hbm.at[pl.ds(off, KD)])
            return carry

        lax.fori_loop(0, DTRIPS, step, 0)

    return pl.kernel(body, out_type=out_type, mesh=mesh, scratch_types=scratch,
                     compiler_params=pltpu.CompilerParams(
                         use_tc_tiling_on_sc=False, needs_layout_passes=False))


_decode_cache = functools.lru_cache(maxsize=None)(_make_decode)


def kernel(x0, x1, edge_index, index, lin0_W, lin0_b, lin1_W, lin1_b,
           W1_rel, W1_root, b1, W2_rel, W2_root, b2, fc_W, fc_b):
    src = jnp.asarray(edge_index[0], _i32)
    dst = jnp.asarray(edge_index[1], _i32)
    i0 = jnp.asarray(index[0], _i32)
    i1 = jnp.asarray(index[1], _i32)

    i0p = jnp.concatenate([i0, jnp.zeros((QP - Q,), _i32)])
    i1p = jnp.concatenate([i1, jnp.zeros((QP - Q,), _i32)])

    t1_0, t1_1, root1 = _m1(
        x0, x1, lin0_W, lin0_b.reshape(1, IN), lin1_W, lin1_b.reshape(1, IN),
        W1_rel, W1_root, b1.reshape(1, HID))
    agg1_0, agg1_1, cnt0, cnt1 = _agg_cache(HID, True)(t1_0, t1_1, src, dst)
    t2p, root2 = _m2(
        root1, agg1_0, agg1_1, cnt0, cnt1, W2_rel, W2_root, b2.reshape(1, OUT))
    agg2_0, agg2_1 = _agg_cache(2 * OUT, False)(t2p, t2p, src, dst)

    wuv = jnp.concatenate([fc_W[:OUT], fc_W[OUT:]], axis=1)
    buv = jnp.concatenate([fc_b, jnp.zeros((1,), _f32)]).reshape(1, 2)
    uv = _m3(root2, agg2_0, agg2_1, cnt0, cnt1, wuv, buv)
    out = _decode_cache()(uv[:, 0], uv[:, 1], i0p, i1p)
    return out[:Q].reshape(Q, 1)
